# baseline (speedup 1.0000x reference)
import jax
import jax.numpy as jnp
from jax import lax
from jax.experimental import pallas as pl
from jax.experimental.pallas import tpu as pltpu
from jax.experimental.pallas import tpu_sc as plsc

E = 320000
D = 128
N = 10000
NW = 32
EPW = E // NW
B = 80
NBLK = EPW // B
NBUF = 8
HI_MASK = -65536


def _decode_body(z_hbm, ei_hbm, out_hbm, src_idx, dst_idx, *rest):
    s_rows = rest[0:2 * NBUF:2]
    d_rows = rest[1:2 * NBUF:2]
    ob = rest[2 * NBUF]
    sems = rest[2 * NBUF + 1:]

    sid = lax.axis_index("s")
    wid = sid * 2 + lax.axis_index("c")
    base = wid * EPW

    pltpu.sync_copy(ei_hbm.at[pl.ds(base, EPW)], src_idx)
    pltpu.sync_copy(ei_hbm.at[pl.ds(E + base, EPW)], dst_idx)

    lanes = lax.iota(jnp.int32, 16)
    last_lane = lanes == 15

    def issue(j, slot):
        pltpu.async_copy(z_hbm.at[src_idx.at[pl.ds(j * B, B)]],
                         s_rows[slot], sems[slot])
        pltpu.async_copy(z_hbm.at[dst_idx.at[pl.ds(j * B, B)]],
                         d_rows[slot], sems[slot])

    def drain(j, slot):
        pltpu.make_async_copy(z_hbm.at[src_idx.at[pl.ds(j * B, B)]],
                              s_rows[slot], sems[slot]).wait()
        pltpu.make_async_copy(z_hbm.at[dst_idx.at[pl.ds(j * B, B)]],
                              d_rows[slot], sems[slot]).wait()

    def widen(row_ref, e, k):
        w = row_ref[e, pl.ds(16 * k, 16)]
        lo = plsc.bitcast(lax.shift_left(w, 16), jnp.float32)
        hi = plsc.bitcast(w & HI_MASK, jnp.float32)
        return lo, hi

    def compute(j, slot):
        sr, dr = s_rows[slot], d_rows[slot]

        def group_body(g, _):
            eb = 16 * g
            for eo in range(16):
                acc = None
                for k in range(D // 32):
                    slo, shi = widen(sr, eb + eo, k)
                    dlo, dhi = widen(dr, eb + eo, k)
                    p = slo * dlo + shi * dhi
                    acc = p if acc is None else acc + p
                tot = plsc.cumsum(acc)
                pos = jnp.full((16,), j * B + eb + eo, jnp.int32)
                plsc.store_scatter(ob, [pos], tot, mask=last_lane)
            return 0

        lax.fori_loop(0, B // 16, group_body, 0)

    for k in range(NBUF - 1):
        issue(k, k)

    def ring_body(i, _):
        for b in range(NBUF):
            j = NBUF * i + b
            drain(j, b)

            @pl.when(j + NBUF - 1 < NBLK)
            def _():
                issue(j + NBUF - 1, (b + NBUF - 1) % NBUF)

            compute(j, b)
        return 0

    lax.fori_loop(0, NBLK // NBUF, ring_body, 0)

    for j in range(NBLK - NBLK % NBUF, NBLK):
        drain(j, j % NBUF)
        compute(j, j % NBUF)

    def sig_body(v, _):
        x = ob[pl.ds(16 * v, 16)]
        ob[pl.ds(16 * v, 16)] = 1.0 / (1.0 + jnp.exp(-x))
        return 0

    lax.fori_loop(0, EPW // 16, sig_body, 0, unroll=8)

    pltpu.sync_copy(ob, out_hbm.at[pl.ds(base, EPW)])


_decode = pl.kernel(
    _decode_body,
    out_type=jax.ShapeDtypeStruct((E,), jnp.float32),
    mesh=plsc.VectorSubcoreMesh(core_axis_name="c", subcore_axis_name="s"),
    scratch_types=(
        [
            pltpu.VMEM((EPW,), jnp.int32),
            pltpu.VMEM((EPW,), jnp.int32),
        ]
        + [pltpu.VMEM((B, D // 2), jnp.int32)] * (2 * NBUF)
        + [pltpu.VMEM((EPW,), jnp.float32)]
        + [pltpu.SemaphoreType.DMA] * NBUF
    ),
    compiler_params=pltpu.CompilerParams(needs_layout_passes=False,
                                         use_tc_tiling_on_sc=False),
)


@jax.jit
def kernel(z, edge_index):
    zb = z.astype(jnp.bfloat16).reshape(N, D // 2, 2)
    z32 = lax.bitcast_convert_type(zb, jnp.int32)
    return _decode(z32, edge_index.astype(jnp.int32).reshape(2 * E))


# speedup vs baseline: 1.0476x; 1.0476x over previous
import jax
import jax.numpy as jnp
from jax import lax
from jax.experimental import pallas as pl
from jax.experimental.pallas import tpu as pltpu
from jax.experimental.pallas import tpu_sc as plsc

E = 320000
D = 128
N = 10000
NW = 32
EPW = E // NW
B = 80
NBLK = EPW // B
NBUF = 8
HI_MASK = -65536


def _decode_body(z_hbm, ei_hbm, out_hbm, src_idx, dst_idx, s_all, d_all,
                 ob, sems):
    sid = lax.axis_index("s")
    wid = sid * 2 + lax.axis_index("c")
    base = wid * EPW

    pltpu.sync_copy(ei_hbm.at[pl.ds(base, EPW)], src_idx)
    pltpu.sync_copy(ei_hbm.at[pl.ds(E + base, EPW)], dst_idx)

    lanes = lax.iota(jnp.int32, 16)
    last_lane = lanes == 15

    def ring(j):
        slot = lax.rem(j, NBUF)
        return slot * B, slot

    def issue(j):
        off, slot = ring(j)
        pltpu.async_copy(z_hbm.at[src_idx.at[pl.ds(j * B, B)]],
                         s_all.at[pl.ds(off, B)], sems.at[slot])
        pltpu.async_copy(z_hbm.at[dst_idx.at[pl.ds(j * B, B)]],
                         d_all.at[pl.ds(off, B)], sems.at[slot])

    def drain(j):
        off, slot = ring(j)
        pltpu.make_async_copy(z_hbm.at[src_idx.at[pl.ds(j * B, B)]],
                              s_all.at[pl.ds(off, B)], sems.at[slot]).wait()
        pltpu.make_async_copy(z_hbm.at[dst_idx.at[pl.ds(j * B, B)]],
                              d_all.at[pl.ds(off, B)], sems.at[slot]).wait()

    def widen(row_ref, e, k):
        w = row_ref[e, pl.ds(16 * k, 16)]
        lo = plsc.bitcast(lax.shift_left(w, 16), jnp.float32)
        hi = plsc.bitcast(w & HI_MASK, jnp.float32)
        return lo, hi

    def compute(j):
        off, _ = ring(j)

        def group_body(g, _):
            eb = off + 16 * g
            for eo in range(16):
                acc = None
                for k in range(D // 32):
                    slo, shi = widen(s_all, eb + eo, k)
                    dlo, dhi = widen(d_all, eb + eo, k)
                    p = slo * dlo + shi * dhi
                    acc = p if acc is None else acc + p
                tot = plsc.cumsum(acc)
                pos = jnp.full((16,), j * B + 16 * g + eo, jnp.int32)
                plsc.store_scatter(ob, [pos], tot, mask=last_lane)
            return 0

        lax.fori_loop(0, B // 16, group_body, 0)

    for k in range(NBUF - 1):
        issue(k)

    def blk_body(j, _):
        drain(j)

        @pl.when(j + NBUF - 1 < NBLK)
        def _():
            issue(j + NBUF - 1)

        compute(j)
        return 0

    lax.fori_loop(0, NBLK, blk_body, 0)

    def sig_body(v, _):
        x = ob[pl.ds(16 * v, 16)]
        ob[pl.ds(16 * v, 16)] = 1.0 / (1.0 + jnp.exp(-x))
        return 0

    lax.fori_loop(0, EPW // 16, sig_body, 0, unroll=8)

    pltpu.sync_copy(ob, out_hbm.at[pl.ds(base, EPW)])


_decode = pl.kernel(
    _decode_body,
    out_type=jax.ShapeDtypeStruct((E,), jnp.float32),
    mesh=plsc.VectorSubcoreMesh(core_axis_name="c", subcore_axis_name="s"),
    scratch_types=[
        pltpu.VMEM((EPW,), jnp.int32),
        pltpu.VMEM((EPW,), jnp.int32),
        pltpu.VMEM((NBUF * B, D // 2), jnp.int32),
        pltpu.VMEM((NBUF * B, D // 2), jnp.int32),
        pltpu.VMEM((EPW,), jnp.float32),
        pltpu.SemaphoreType.DMA((NBUF,)),
    ],
    compiler_params=pltpu.CompilerParams(needs_layout_passes=False,
                                         use_tc_tiling_on_sc=False),
)


@jax.jit
def kernel(z, edge_index):
    zb = z.astype(jnp.bfloat16).reshape(N, D // 2, 2)
    z32 = lax.bitcast_convert_type(zb, jnp.int32)
    return _decode(z32, edge_index.astype(jnp.int32).reshape(2 * E))


# speedup vs baseline: 1.1621x; 1.1092x over previous
import jax
import jax.numpy as jnp
from jax import lax
from jax.experimental import pallas as pl
from jax.experimental.pallas import tpu as pltpu
from jax.experimental.pallas import tpu_sc as plsc

E = 320000
D = 128
N = 10000
NW = 32
EPW = E // NW
B = 80
NBLK = EPW // B
NBUF = 4
HI_MASK = -65536


def _decode_body(z_hbm, ei_hbm, out_hbm, src_idx, dst_idx, s_all, d_all,
                 ob, sems):
    sid = lax.axis_index("s")
    wid = sid * 2 + lax.axis_index("c")
    base = wid * EPW

    pltpu.sync_copy(ei_hbm.at[pl.ds(base, EPW)], src_idx)
    pltpu.sync_copy(ei_hbm.at[pl.ds(E + base, EPW)], dst_idx)

    lanes = lax.iota(jnp.int32, 16)
    last_lane = lanes == 15

    def ring(j):
        slot = lax.rem(j, NBUF)
        return slot * B, slot

    def issue(j):
        off, slot = ring(j)
        pltpu.async_copy(z_hbm.at[src_idx.at[pl.ds(j * B, B)]],
                         s_all.at[pl.ds(off, B)], sems.at[slot])
        pltpu.async_copy(z_hbm.at[dst_idx.at[pl.ds(j * B, B)]],
                         d_all.at[pl.ds(off, B)], sems.at[slot])

    def drain(j):
        off, slot = ring(j)
        pltpu.make_async_copy(z_hbm.at[src_idx.at[pl.ds(j * B, B)]],
                              s_all.at[pl.ds(off, B)], sems.at[slot]).wait()
        pltpu.make_async_copy(z_hbm.at[dst_idx.at[pl.ds(j * B, B)]],
                              d_all.at[pl.ds(off, B)], sems.at[slot]).wait()

    def compute(j):
        off, _ = ring(j)

        def group_body(g, _):
            eb = off + 16 * g
            for eo in range(16):
                acc = None
                for k in range(D // 16):
                    p = (s_all[eb + eo, pl.ds(16 * k, 16)]
                         * d_all[eb + eo, pl.ds(16 * k, 16)])
                    acc = p if acc is None else acc + p
                tot = plsc.cumsum(acc)
                pos = jnp.full((16,), j * B + 16 * g + eo, jnp.int32)
                plsc.store_scatter(ob, [pos], tot, mask=last_lane)
            return 0

        lax.fori_loop(0, B // 16, group_body, 0)

    for k in range(NBUF - 1):
        issue(k)

    def blk_body(j, _):
        drain(j)

        @pl.when(j + NBUF - 1 < NBLK)
        def _():
            issue(j + NBUF - 1)

        compute(j)
        return 0

    lax.fori_loop(0, NBLK, blk_body, 0)

    def sig_body(v, _):
        x = ob[pl.ds(16 * v, 16)]
        ob[pl.ds(16 * v, 16)] = 1.0 / (1.0 + jnp.exp(-x))
        return 0

    lax.fori_loop(0, EPW // 16, sig_body, 0, unroll=8)

    pltpu.sync_copy(ob, out_hbm.at[pl.ds(base, EPW)])


_decode = pl.kernel(
    _decode_body,
    out_type=jax.ShapeDtypeStruct((E,), jnp.float32),
    mesh=plsc.VectorSubcoreMesh(core_axis_name="c", subcore_axis_name="s"),
    scratch_types=[
        pltpu.VMEM((EPW,), jnp.int32),
        pltpu.VMEM((EPW,), jnp.int32),
        pltpu.VMEM((NBUF * B, D), jnp.float32),
        pltpu.VMEM((NBUF * B, D), jnp.float32),
        pltpu.VMEM((EPW,), jnp.float32),
        pltpu.SemaphoreType.DMA((NBUF,)),
    ],
    compiler_params=pltpu.CompilerParams(needs_layout_passes=False),
)


@jax.jit
def kernel(z, edge_index):
    return _decode(z, edge_index.astype(jnp.int32).reshape(2 * E))


# speedup vs baseline: 1.1630x; 1.0008x over previous
import jax
import jax.numpy as jnp
from jax import lax
from jax.experimental import pallas as pl
from jax.experimental.pallas import tpu as pltpu
from jax.experimental.pallas import tpu_sc as plsc

E = 320000
D = 128
N = 10000
NW = 32
EPW = E // NW
B = 80
NBLK = EPW // B
NBUF = 4


def _decode_body(z_hbm, ei_hbm, out_hbm, src_idx, dst_idx, s_all, d_all,
                 ob, sems):
    sid = lax.axis_index("s")
    wid = sid * 2 + lax.axis_index("c")
    base = wid * EPW

    pltpu.sync_copy(ei_hbm.at[pl.ds(base, EPW)], src_idx)
    pltpu.sync_copy(ei_hbm.at[pl.ds(E + base, EPW)], dst_idx)

    lanes = lax.iota(jnp.int32, 16)
    last_lane = lanes == 15

    def ring(j):
        slot = lax.rem(j, NBUF)
        return slot * B, slot

    def issue(j):
        off, slot = ring(j)
        pltpu.async_copy(z_hbm.at[src_idx.at[pl.ds(j * B, B)]],
                         s_all.at[pl.ds(off, B)], sems.at[slot])
        pltpu.async_copy(z_hbm.at[dst_idx.at[pl.ds(j * B, B)]],
                         d_all.at[pl.ds(off, B)], sems.at[slot])

    def drain(j):
        off, slot = ring(j)
        pltpu.make_async_copy(z_hbm.at[src_idx.at[pl.ds(j * B, B)]],
                              s_all.at[pl.ds(off, B)], sems.at[slot]).wait()
        pltpu.make_async_copy(z_hbm.at[dst_idx.at[pl.ds(j * B, B)]],
                              d_all.at[pl.ds(off, B)], sems.at[slot]).wait()

    def compute(j):
        off, _ = ring(j)

        def group_body(g, _):
            eb = off + 16 * g
            for eo in range(16):
                acc = None
                for k in range(D // 16):
                    p = (s_all[eb + eo, pl.ds(16 * k, 16)]
                         * d_all[eb + eo, pl.ds(16 * k, 16)])
                    acc = p if acc is None else acc + p
                tot = plsc.cumsum(acc)
                pos = jnp.full((16,), j * B + 16 * g + eo, jnp.int32)
                plsc.store_scatter(ob, [pos], tot, mask=last_lane)
            return 0

        lax.fori_loop(0, B // 16, group_body, 0)

    for k in range(NBUF - 1):
        issue(k)

    def blk_body(j, _):
        drain(j)

        @pl.when(j + NBUF - 1 < NBLK)
        def _():
            issue(j + NBUF - 1)

        compute(j)
        return 0

    lax.fori_loop(0, NBLK, blk_body, 0)

    def sig_body(v, _):
        x = ob[pl.ds(16 * v, 16)]
        ob[pl.ds(16 * v, 16)] = 1.0 / (1.0 + jnp.exp(-x))
        return 0

    lax.fori_loop(0, EPW // 16, sig_body, 0, unroll=8)

    pltpu.sync_copy(ob, out_hbm.at[pl.ds(base, EPW)])


_decode = pl.kernel(
    _decode_body,
    out_type=jax.ShapeDtypeStruct((E,), jnp.float32),
    mesh=plsc.VectorSubcoreMesh(core_axis_name="c", subcore_axis_name="s"),
    scratch_types=[
        pltpu.VMEM((EPW,), jnp.int32),
        pltpu.VMEM((EPW,), jnp.int32),
        pltpu.VMEM((NBUF * B, D), jnp.float32),
        pltpu.VMEM((NBUF * B, D), jnp.float32),
        pltpu.VMEM((EPW,), jnp.float32),
        pltpu.SemaphoreType.DMA((NBUF,)),
    ],
    compiler_params=pltpu.CompilerParams(needs_layout_passes=False),
)


@jax.jit
def kernel(z, edge_index):
    return _decode(z, edge_index.astype(jnp.int32).reshape(2 * E))
